# baseline (speedup 1.0000x reference)
import functools

import jax
import jax.numpy as jnp
from jax import lax
from jax.experimental import pallas as pl
from jax.experimental.pallas import tpu as pltpu
from jax.experimental.pallas import tpu_sc as plsc

N = 10000
E = 320000
DN = 128
DE = 16
DG = 16
L = 128

NC = 2
NS = 16
NW = NC * NS

CH = 80
CPW = E // (NW * CH)
NCHUNK = E // CH
ROWS_PER_TILE = N // NS

@functools.lru_cache(maxsize=None)
def _sc_mesh():
    return plsc.VectorSubcoreMesh(
        core_axis_name="c", subcore_axis_name="s", num_cores=NC, num_subcores=NS)


def _pack_bf16_pairs(v):
    lo = jax.lax.bitcast_convert_type(v[:, :64].astype(jnp.bfloat16), jnp.int16)
    hi = jax.lax.bitcast_convert_type(v[:, 64:].astype(jnp.bfloat16), jnp.int16)
    return (hi.astype(jnp.int32) << 16) | (lo.astype(jnp.int32) & 0xFFFF)


def _unpack_bf16_pairs(w):
    lo = jax.lax.bitcast_convert_type(w << 16, jnp.float32)
    hi = jax.lax.bitcast_convert_type(w & jnp.int32(-65536), jnp.float32)
    return jnp.concatenate([lo, hi], axis=1)


def _tables_body(x_ref, ws_ref, wd_ref, u_ref, weu_ref, be1_ref, xs_ref, xd_ref):
    xb = x_ref[...]
    xs_ref[...] = _pack_bf16_pairs(
        jnp.dot(xb, ws_ref[...], preferred_element_type=jnp.float32))
    c0 = jnp.dot(u_ref[...], weu_ref[...], preferred_element_type=jnp.float32) + be1_ref[...]
    xd_ref[...] = _pack_bf16_pairs(
        jnp.dot(xb, wd_ref[...], preferred_element_type=jnp.float32) + c0)


def _node_tables(x, ws, wd, u, weu, be1):
    bn = 2000
    return pl.pallas_call(
        _tables_body,
        grid=(N // bn,),
        in_specs=[
            pl.BlockSpec((bn, DN), lambda i: (i, 0)),
            pl.BlockSpec((DN, L), lambda i: (0, 0)),
            pl.BlockSpec((DN, L), lambda i: (0, 0)),
            pl.BlockSpec((1, DG), lambda i: (0, 0)),
            pl.BlockSpec((DG, L), lambda i: (0, 0)),
            pl.BlockSpec((1, L), lambda i: (0, 0)),
        ],
        out_specs=[
            pl.BlockSpec((bn, L // 2), lambda i: (i, 0)),
            pl.BlockSpec((bn, L // 2), lambda i: (i, 0)),
        ],
        out_shape=[
            jax.ShapeDtypeStruct((N, L // 2), jnp.int32),
            jax.ShapeDtypeStruct((N, L // 2), jnp.int32),
        ],
    )(x, ws, wd, u, weu, be1)


_NSLOT = 4


def _gather_body(xs_hbm, xd_hbm, row_hbm, col_hbm, gs_hbm, gd_hbm,
                 idxr, idxc, bufa, bufb, ga, gb, wa, wb):
    c = lax.axis_index("c")
    s = lax.axis_index("s")
    base = c * (NCHUNK // 2) + s * CPW
    lane0 = c * (L // 2)
    pltpu.sync_copy(row_hbm.at[pl.ds(base, CPW)], idxr)
    pltpu.sync_copy(col_hbm.at[pl.ds(base, CPW)], idxc)

    def start_gather(p, j):
        pltpu.async_copy(xs_hbm.at[idxr.at[j]], bufa.at[p], ga.at[p])
        pltpu.async_copy(xd_hbm.at[idxc.at[j]], bufb.at[p], gb.at[p])

    def wait_gather(p):
        pltpu.make_async_copy(xs_hbm.at[idxr.at[0]], bufa.at[p], ga.at[p]).wait()
        pltpu.make_async_copy(xd_hbm.at[idxc.at[0]], bufb.at[p], gb.at[p]).wait()

    def dst(hbm, j):
        r0 = s * (E // 2 // NS) + j * CH
        return hbm.at[pl.ds(r0, CH), pl.ds(lane0, L // 2)]

    def start_write(p, j):
        pltpu.async_copy(bufa.at[p], dst(gs_hbm, j), wa.at[p])
        pltpu.async_copy(bufb.at[p], dst(gd_hbm, j), wb.at[p])

    def wait_write(p):
        pltpu.make_async_copy(bufa.at[p], dst(gs_hbm, 0), wa.at[p]).wait()
        pltpu.make_async_copy(bufb.at[p], dst(gd_hbm, 0), wb.at[p]).wait()

    for p in range(_NSLOT):
        start_gather(p, p)

    def body(j4, carry):
        for p in range(_NSLOT):
            j = j4 * _NSLOT + p
            wait_gather(p)
            start_write(p, j)
        for p in range(_NSLOT):
            jn = j4 * _NSLOT + p + _NSLOT
            wait_write(p)

            @pl.when(jn < CPW)
            def _():
                start_gather(p, jn)
        return carry

    lax.fori_loop(0, CPW // _NSLOT, body, 0)
    for j in range((CPW // _NSLOT) * _NSLOT, CPW):
        p = j % _NSLOT
        wait_gather(p)
        start_write(p, j)
        wait_write(p)


@functools.lru_cache(maxsize=None)
def _gather_kernel_fn():
    return pl.kernel(
        _gather_body,
        out_type=(
            jax.ShapeDtypeStruct((E // 2, L), jnp.int32),
            jax.ShapeDtypeStruct((E // 2, L), jnp.int32),
        ),
        mesh=_sc_mesh(),
        scratch_types=[
            pltpu.VMEM((CPW, CH), jnp.int32),
            pltpu.VMEM((CPW, CH), jnp.int32),
            pltpu.VMEM((_NSLOT, CH, L // 2), jnp.int32),
            pltpu.VMEM((_NSLOT, CH, L // 2), jnp.int32),
            pltpu.SemaphoreType.DMA((_NSLOT,)),
            pltpu.SemaphoreType.DMA((_NSLOT,)),
            pltpu.SemaphoreType.DMA((_NSLOT,)),
            pltpu.SemaphoreType.DMA((_NSLOT,)),
        ],
        compiler_params=pltpu.CompilerParams(use_tc_tiling_on_sc=False),
    )


def _gather_kernel(xs, xd, row2d, col2d):
    return _gather_kernel_fn()(xs, xd, row2d, col2d)


_BH = 3200
_NBLK = (E // 2) // _BH


def _edge_body(gs_ref, gd_ref, eat_ref, wee_ref, we2_ref, be2_ref, out_ref):
    in_hi_half = pl.program_id(0) >= _NBLK
    half = L // 2

    def unpack(w):
        lo = jax.lax.bitcast_convert_type(w << 16, jnp.float32)
        hi = jax.lax.bitcast_convert_type(w & jnp.int32(-65536), jnp.float32)
        return jnp.concatenate(
            [jnp.where(in_hi_half, lo[:, half:], lo[:, :half]),
             jnp.where(in_hi_half, hi[:, half:], hi[:, :half])], axis=1)

    g = unpack(gs_ref[...]) + unpack(gd_ref[...])
    eaw = jax.lax.dot_general(
        eat_ref[...], wee_ref[...], (((0,), (0,)), ((), ())),
        preferred_element_type=jnp.float32)
    rh = jnp.maximum(g + eaw, 0.0)
    out_ref[...] = jax.lax.dot_general(
        we2_ref[...], rh, (((0,), (1,)), ((), ())),
        preferred_element_type=jnp.float32) + be2_ref[...]


def _edge_mlp(gs, gd, ea_t, wee, we2, be2):
    return pl.pallas_call(
        _edge_body,
        grid=(2 * _NBLK,),
        in_specs=[
            pl.BlockSpec((_BH, L), lambda i: (i % _NBLK, 0)),
            pl.BlockSpec((_BH, L), lambda i: (i % _NBLK, 0)),
            pl.BlockSpec((DE, _BH), lambda i: (0, i)),
            pl.BlockSpec((DE, L), lambda i: (0, 0)),
            pl.BlockSpec((L, DE), lambda i: (0, 0)),
            pl.BlockSpec((DE, 1), lambda i: (0, 0)),
        ],
        out_specs=pl.BlockSpec((DE, _BH), lambda i: (0, i)),
        out_shape=jax.ShapeDtypeStruct((DE, E), jnp.float32),
    )(gs, gd, ea_t, wee, we2, be2)


def _scatter_body(eot_hbm, col_hbm, zeros_hbm, out_hbm, idxc, tbuf0, tbuf1,
                  vals0, vals1, agg, tl, ta):
    tbufs = (tbuf0, tbuf1)
    valss = (vals0, vals1)
    c = lax.axis_index("c")
    s = lax.axis_index("s")
    pltpu.sync_copy(zeros_hbm, agg.at[pl.ds(s * ROWS_PER_TILE, ROWS_PER_TILE)])
    plsc.subcore_barrier()

    base = (c * NS + s) * CPW
    pltpu.sync_copy(col_hbm.at[pl.ds(base, CPW)], idxc)
    rows16 = jax.lax.iota(jnp.int32, DE)

    def start_load(p, j):
        e0 = (base + j) * CH
        pltpu.async_copy(eot_hbm.at[:, pl.ds(e0, CH)], tbufs[p], tl.at[p])

    def wait_load(p):
        pltpu.make_async_copy(eot_hbm.at[:, pl.ds(0, CH)], tbufs[p],
                              tl.at[p]).wait()

    def start_add(p, j):
        pltpu.async_copy(valss[p], agg.at[idxc.at[j]], ta.at[p], add=True)

    def wait_add(p):
        pltpu.make_async_copy(valss[p], agg.at[idxc.at[0]], ta.at[p]).wait()

    def transpose(p):
        def tblock(j16, cc):
            for u in range(16):
                e = j16 * 16 + u
                v = plsc.load_gather(tbufs[p],
                                     [rows16, jnp.full((DE,), 0, jnp.int32) + e])
                valss[p][e, pl.ds(0, DE)] = v
            return cc

        lax.fori_loop(0, CH // 16, tblock, 0)

    start_load(0, 0)

    def body(j2, carry):
        for p in range(2):
            j = 2 * j2 + p
            start_load((p + 1) % 2, j + 1)
            wait_load(p)

            @pl.when(j >= 2)
            def _():
                wait_add(p)

            transpose(p)
            start_add(p, j)
        return carry

    lax.fori_loop(0, (CPW - 1) // 2, body, 0)
    wait_load(0)
    wait_add(0)
    transpose(0)
    start_add(0, CPW - 1)
    wait_add(1)
    wait_add(0)
    plsc.subcore_barrier()
    r0 = s * ROWS_PER_TILE
    pltpu.sync_copy(agg.at[pl.ds(r0, ROWS_PER_TILE)],
                    out_hbm.at[c].at[pl.ds(r0, ROWS_PER_TILE)])


@functools.lru_cache(maxsize=None)
def _scatter_kernel_fn():
    return pl.kernel(
        _scatter_body,
        out_type=jax.ShapeDtypeStruct((NC, N, DE), jnp.float32),
        mesh=_sc_mesh(),
        scratch_types=[
            pltpu.VMEM((CPW, CH), jnp.int32),
            pltpu.VMEM((DE, CH), jnp.float32),
            pltpu.VMEM((DE, CH), jnp.float32),
            pltpu.VMEM((CH, DE), jnp.float32),
            pltpu.VMEM((CH, DE), jnp.float32),
            pltpu.VMEM_SHARED((N, DE), jnp.float32),
            pltpu.SemaphoreType.DMA((2,)),
            pltpu.SemaphoreType.DMA((2,)),
        ],
        compiler_params=pltpu.CompilerParams(
            use_tc_tiling_on_sc=False, needs_layout_passes=False),
    )


def _scatter_kernel(eo_t, col2d, zeros):
    return _scatter_kernel_fn()(eo_t, col2d, zeros)


def _node_body(x_ref, p_ref, u_ref, wnx_ref, wna_ref, wnu_ref, bn1_ref,
               wn2_ref, bn2_ref, wgu_ref, wgm_ref, bg1_ref, wg2_ref, bg2_ref,
               xo_ref, go_ref, acc_ref):
    i = pl.program_id(0)

    @pl.when(i == 0)
    def _init():
        acc_ref[...] = jnp.zeros_like(acc_ref)

    agg = p_ref[0] + p_ref[1]
    cn = jnp.dot(u_ref[...], wnu_ref[...], preferred_element_type=jnp.float32) + bn1_ref[...]
    nh = jnp.maximum(
        jnp.dot(x_ref[...], wnx_ref[...], preferred_element_type=jnp.float32)
        + jnp.dot(agg, wna_ref[...], preferred_element_type=jnp.float32) + cn,
        0.0)
    xo = jnp.dot(nh, wn2_ref[...], preferred_element_type=jnp.float32) + bn2_ref[...]
    xo_ref[...] = xo
    acc_ref[...] += jnp.sum(xo, axis=0, keepdims=True)

    @pl.when(i == pl.num_programs(0) - 1)
    def _globals():
        mean = acc_ref[...] * (1.0 / N)
        gh = jnp.maximum(
            jnp.dot(u_ref[...], wgu_ref[...], preferred_element_type=jnp.float32)
            + jnp.dot(mean, wgm_ref[...], preferred_element_type=jnp.float32)
            + bg1_ref[...],
            0.0)
        go_ref[...] = jnp.dot(gh, wg2_ref[...], preferred_element_type=jnp.float32) + bg2_ref[...]


def _node_global(x, p, u, wnx, wna, wnu, bn1, wn2, bn2, wgu, wgm, bg1, wg2, bg2):
    bn = 2000
    return pl.pallas_call(
        _node_body,
        grid=(N // bn,),
        in_specs=[
            pl.BlockSpec((bn, DN), lambda i: (i, 0)),
            pl.BlockSpec((NC, bn, DE), lambda i: (0, i, 0)),
            pl.BlockSpec((1, DG), lambda i: (0, 0)),
            pl.BlockSpec((DN, L), lambda i: (0, 0)),
            pl.BlockSpec((DE, L), lambda i: (0, 0)),
            pl.BlockSpec((DG, L), lambda i: (0, 0)),
            pl.BlockSpec((1, L), lambda i: (0, 0)),
            pl.BlockSpec((L, DN), lambda i: (0, 0)),
            pl.BlockSpec((1, DN), lambda i: (0, 0)),
            pl.BlockSpec((DG, L), lambda i: (0, 0)),
            pl.BlockSpec((DN, L), lambda i: (0, 0)),
            pl.BlockSpec((1, L), lambda i: (0, 0)),
            pl.BlockSpec((L, DG), lambda i: (0, 0)),
            pl.BlockSpec((1, DG), lambda i: (0, 0)),
        ],
        out_specs=[
            pl.BlockSpec((bn, DN), lambda i: (i, 0)),
            pl.BlockSpec((1, DG), lambda i: (0, 0)),
        ],
        out_shape=[
            jax.ShapeDtypeStruct((N, DN), jnp.float32),
            jax.ShapeDtypeStruct((1, DG), jnp.float32),
        ],
        scratch_shapes=[pltpu.VMEM((1, DN), jnp.float32)],
    )(x, p, u, wnx, wna, wnu, bn1, wn2, bn2, wgu, wgm, bg1, wg2, bg2)


def kernel(x, edge_index, edge_attr, global_attr,
           We1, be1, We2, be2,
           Wn1, bn1, Wn2, bn2,
           Wg1, bg1, Wg2, bg2):
    row2d = edge_index[0].reshape(NCHUNK, CH)
    col2d = edge_index[1].reshape(NCHUNK, CH)

    xs, xd = _node_tables(
        x, We1[:DN], We1[DN:2 * DN], global_attr,
        We1[2 * DN + DE:], be1.reshape(1, L))

    gs, gd = _gather_kernel(xs, xd, row2d, col2d)

    eo_t = _edge_mlp(
        gs, gd, edge_attr.T, We1[2 * DN:2 * DN + DE], We2,
        be2.reshape(DE, 1))
    edge_out = eo_t.T

    zeros = jnp.zeros((ROWS_PER_TILE, DE), jnp.float32)
    partials = _scatter_kernel(eo_t, col2d, zeros)

    x_out, global_out = _node_global(
        x, partials, global_attr,
        Wn1[:DN], Wn1[DN:DN + DE], Wn1[DN + DE:], bn1.reshape(1, L),
        Wn2, bn2.reshape(1, DN),
        Wg1[:DG], Wg1[DG:], bg1.reshape(1, L),
        Wg2, bg2.reshape(1, DG))

    return (x_out, edge_out, global_out)


# speedup vs baseline: 1.0008x; 1.0008x over previous
import functools

import jax
import jax.numpy as jnp
from jax import lax
from jax.experimental import pallas as pl
from jax.experimental.pallas import tpu as pltpu
from jax.experimental.pallas import tpu_sc as plsc

N = 10000
E = 320000
DN = 128
DE = 16
DG = 16
L = 128

NC = 2
NS = 16
NW = NC * NS

CH = 80
CPW = E // (NW * CH)
NCHUNK = E // CH
ROWS_PER_TILE = N // NS

@functools.lru_cache(maxsize=None)
def _sc_mesh():
    return plsc.VectorSubcoreMesh(
        core_axis_name="c", subcore_axis_name="s", num_cores=NC, num_subcores=NS)


def _pack_bf16_pairs(v):
    lo = jax.lax.bitcast_convert_type(v[:, :64].astype(jnp.bfloat16), jnp.int16)
    hi = jax.lax.bitcast_convert_type(v[:, 64:].astype(jnp.bfloat16), jnp.int16)
    return (hi.astype(jnp.int32) << 16) | (lo.astype(jnp.int32) & 0xFFFF)


def _tables_body(x_ref, ws_ref, wd_ref, u_ref, weu_ref, be1_ref, xs_ref, xd_ref):
    xb = x_ref[...]
    xs_ref[...] = _pack_bf16_pairs(
        jnp.dot(xb, ws_ref[...], preferred_element_type=jnp.float32))
    c0 = jnp.dot(u_ref[...], weu_ref[...], preferred_element_type=jnp.float32) + be1_ref[...]
    xd_ref[...] = _pack_bf16_pairs(
        jnp.dot(xb, wd_ref[...], preferred_element_type=jnp.float32) + c0)


def _node_tables(x, ws, wd, u, weu, be1):
    bn = 2000
    return pl.pallas_call(
        _tables_body,
        grid=(N // bn,),
        in_specs=[
            pl.BlockSpec((bn, DN), lambda i: (i, 0)),
            pl.BlockSpec((DN, L), lambda i: (0, 0)),
            pl.BlockSpec((DN, L), lambda i: (0, 0)),
            pl.BlockSpec((1, DG), lambda i: (0, 0)),
            pl.BlockSpec((DG, L), lambda i: (0, 0)),
            pl.BlockSpec((1, L), lambda i: (0, 0)),
        ],
        out_specs=[
            pl.BlockSpec((bn, L // 2), lambda i: (i, 0)),
            pl.BlockSpec((bn, L // 2), lambda i: (i, 0)),
        ],
        out_shape=[
            jax.ShapeDtypeStruct((N, L // 2), jnp.int32),
            jax.ShapeDtypeStruct((N, L // 2), jnp.int32),
        ],
    )(x, ws, wd, u, weu, be1)


_NSLOT = 4


def _gather_body(xs_hbm, xd_hbm, row_hbm, col_hbm, gs_hbm, gd_hbm,
                 idxr, idxc, bufa, bufb, ga, gb, wa, wb):
    c = lax.axis_index("c")
    s = lax.axis_index("s")
    base = c * (NCHUNK // 2) + s * CPW
    lane0 = c * (L // 2)
    pltpu.sync_copy(row_hbm.at[pl.ds(base, CPW)], idxr)
    pltpu.sync_copy(col_hbm.at[pl.ds(base, CPW)], idxc)

    def start_gather(p, j):
        pltpu.async_copy(xs_hbm.at[idxr.at[j]], bufa.at[p], ga.at[p])
        pltpu.async_copy(xd_hbm.at[idxc.at[j]], bufb.at[p], gb.at[p])

    def wait_gather(p):
        pltpu.make_async_copy(xs_hbm.at[idxr.at[0]], bufa.at[p], ga.at[p]).wait()
        pltpu.make_async_copy(xd_hbm.at[idxc.at[0]], bufb.at[p], gb.at[p]).wait()

    def dst(hbm, j):
        r0 = s * (E // 2 // NS) + j * CH
        return hbm.at[pl.ds(r0, CH), pl.ds(lane0, L // 2)]

    def start_write(p, j):
        pltpu.async_copy(bufa.at[p], dst(gs_hbm, j), wa.at[p])
        pltpu.async_copy(bufb.at[p], dst(gd_hbm, j), wb.at[p])

    def wait_write(p):
        pltpu.make_async_copy(bufa.at[p], dst(gs_hbm, 0), wa.at[p]).wait()
        pltpu.make_async_copy(bufb.at[p], dst(gd_hbm, 0), wb.at[p]).wait()

    for p in range(_NSLOT):
        start_gather(p, p)

    def body(j4, carry):
        for p in range(_NSLOT):
            j = j4 * _NSLOT + p
            wait_gather(p)
            start_write(p, j)
        for p in range(_NSLOT):
            jn = j4 * _NSLOT + p + _NSLOT
            wait_write(p)

            @pl.when(jn < CPW)
            def _():
                start_gather(p, jn)
        return carry

    lax.fori_loop(0, CPW // _NSLOT, body, 0)
    for j in range((CPW // _NSLOT) * _NSLOT, CPW):
        p = j % _NSLOT
        wait_gather(p)
        start_write(p, j)
        wait_write(p)


@functools.lru_cache(maxsize=None)
def _gather_kernel_fn():
    return pl.kernel(
        _gather_body,
        out_type=(
            jax.ShapeDtypeStruct((E // 2, L), jnp.int32),
            jax.ShapeDtypeStruct((E // 2, L), jnp.int32),
        ),
        mesh=_sc_mesh(),
        scratch_types=[
            pltpu.VMEM((CPW, CH), jnp.int32),
            pltpu.VMEM((CPW, CH), jnp.int32),
            pltpu.VMEM((_NSLOT, CH, L // 2), jnp.int32),
            pltpu.VMEM((_NSLOT, CH, L // 2), jnp.int32),
            pltpu.SemaphoreType.DMA((_NSLOT,)),
            pltpu.SemaphoreType.DMA((_NSLOT,)),
            pltpu.SemaphoreType.DMA((_NSLOT,)),
            pltpu.SemaphoreType.DMA((_NSLOT,)),
        ],
        compiler_params=pltpu.CompilerParams(use_tc_tiling_on_sc=False),
    )


def _gather_kernel(xs, xd, row2d, col2d):
    return _gather_kernel_fn()(xs, xd, row2d, col2d)


_BH = 3200
_NBLK = (E // 2) // _BH


def _edge_body(gs_ref, gd_ref, eat_ref, wee_ref, we2_ref, be2_ref, out_ref):
    in_hi_half = pl.program_id(0) >= _NBLK
    half = L // 2

    def unpack(w):
        lo = jax.lax.bitcast_convert_type(w << 16, jnp.float32)
        hi = jax.lax.bitcast_convert_type(w & jnp.int32(-65536), jnp.float32)
        return jnp.concatenate(
            [jnp.where(in_hi_half, lo[:, half:], lo[:, :half]),
             jnp.where(in_hi_half, hi[:, half:], hi[:, :half])], axis=1)

    g = unpack(gs_ref[...]) + unpack(gd_ref[...])
    eaw = jax.lax.dot_general(
        eat_ref[...], wee_ref[...], (((0,), (0,)), ((), ())),
        preferred_element_type=jnp.float32)
    rh = jnp.maximum(g + eaw, 0.0)
    out_ref[...] = jax.lax.dot_general(
        we2_ref[...], rh, (((0,), (1,)), ((), ())),
        preferred_element_type=jnp.float32) + be2_ref[...]


def _edge_mlp(gs, gd, ea_t, wee, we2, be2):
    return pl.pallas_call(
        _edge_body,
        grid=(2 * _NBLK,),
        in_specs=[
            pl.BlockSpec((_BH, L), lambda i: (i % _NBLK, 0)),
            pl.BlockSpec((_BH, L), lambda i: (i % _NBLK, 0)),
            pl.BlockSpec((DE, _BH), lambda i: (0, i)),
            pl.BlockSpec((DE, L), lambda i: (0, 0)),
            pl.BlockSpec((L, DE), lambda i: (0, 0)),
            pl.BlockSpec((DE, 1), lambda i: (0, 0)),
        ],
        out_specs=pl.BlockSpec((DE, _BH), lambda i: (0, i)),
        out_shape=jax.ShapeDtypeStruct((DE, E), jnp.float32),
    )(gs, gd, ea_t, wee, we2, be2)


def _scatter_body(eot_hbm, col_hbm, zeros_hbm, out_hbm, idxc, tbuf0, tbuf1,
                  vals0, vals1, agg, tl, ta):
    tbufs = (tbuf0, tbuf1)
    valss = (vals0, vals1)
    c = lax.axis_index("c")
    s = lax.axis_index("s")
    pltpu.sync_copy(zeros_hbm, agg.at[pl.ds(s * ROWS_PER_TILE, ROWS_PER_TILE)])
    plsc.subcore_barrier()

    base = (c * NS + s) * CPW
    pltpu.sync_copy(col_hbm.at[pl.ds(base, CPW)], idxc)
    rows16 = jax.lax.iota(jnp.int32, DE)

    def start_load(p, j):
        e0 = (base + j) * CH
        pltpu.async_copy(eot_hbm.at[:, pl.ds(e0, CH)], tbufs[p], tl.at[p])

    def wait_load(p):
        pltpu.make_async_copy(eot_hbm.at[:, pl.ds(0, CH)], tbufs[p],
                              tl.at[p]).wait()

    def start_add(p, j):
        pltpu.async_copy(valss[p], agg.at[idxc.at[j]], ta.at[p], add=True)

    def wait_add(p):
        pltpu.make_async_copy(valss[p], agg.at[idxc.at[0]], ta.at[p]).wait()

    def transpose(p):
        def tblock(j16, cc):
            for u in range(16):
                e = j16 * 16 + u
                v = plsc.load_gather(tbufs[p],
                                     [rows16, jnp.full((DE,), 0, jnp.int32) + e])
                valss[p][e, pl.ds(0, DE)] = v
            return cc

        lax.fori_loop(0, CH // 16, tblock, 0)

    start_load(0, 0)

    def body(j2, carry):
        for p in range(2):
            j = 2 * j2 + p
            start_load((p + 1) % 2, j + 1)
            wait_load(p)

            @pl.when(j >= 2)
            def _():
                wait_add(p)

            transpose(p)
            start_add(p, j)
        return carry

    lax.fori_loop(0, (CPW - 1) // 2, body, 0)
    wait_load(0)
    wait_add(0)
    transpose(0)
    start_add(0, CPW - 1)
    wait_add(1)
    wait_add(0)
    plsc.subcore_barrier()
    r0 = s * ROWS_PER_TILE
    pltpu.sync_copy(agg.at[pl.ds(r0, ROWS_PER_TILE)],
                    out_hbm.at[c].at[pl.ds(r0, ROWS_PER_TILE)])


@functools.lru_cache(maxsize=None)
def _scatter_kernel_fn():
    return pl.kernel(
        _scatter_body,
        out_type=jax.ShapeDtypeStruct((NC, N, DE), jnp.float32),
        mesh=_sc_mesh(),
        scratch_types=[
            pltpu.VMEM((CPW, CH), jnp.int32),
            pltpu.VMEM((DE, CH), jnp.float32),
            pltpu.VMEM((DE, CH), jnp.float32),
            pltpu.VMEM((CH, DE), jnp.float32),
            pltpu.VMEM((CH, DE), jnp.float32),
            pltpu.VMEM_SHARED((N, DE), jnp.float32),
            pltpu.SemaphoreType.DMA((2,)),
            pltpu.SemaphoreType.DMA((2,)),
        ],
        compiler_params=pltpu.CompilerParams(
            use_tc_tiling_on_sc=False, needs_layout_passes=False),
    )


def _scatter_kernel(eo_t, col2d, zeros):
    return _scatter_kernel_fn()(eo_t, col2d, zeros)


def _node_body(x_ref, p_ref, u_ref, wnx_ref, wna_ref, wnu_ref, bn1_ref,
               wn2_ref, bn2_ref, wgu_ref, wgm_ref, bg1_ref, wg2_ref, bg2_ref,
               xo_ref, go_ref, acc_ref):
    i = pl.program_id(0)

    @pl.when(i == 0)
    def _init():
        acc_ref[...] = jnp.zeros_like(acc_ref)

    agg = p_ref[0] + p_ref[1]
    cn = jnp.dot(u_ref[...], wnu_ref[...], preferred_element_type=jnp.float32) + bn1_ref[...]
    nh = jnp.maximum(
        jnp.dot(x_ref[...], wnx_ref[...], preferred_element_type=jnp.float32)
        + jnp.dot(agg, wna_ref[...], preferred_element_type=jnp.float32) + cn,
        0.0)
    xo = jnp.dot(nh, wn2_ref[...], preferred_element_type=jnp.float32) + bn2_ref[...]
    xo_ref[...] = xo
    acc_ref[...] += jnp.sum(xo, axis=0, keepdims=True)

    @pl.when(i == pl.num_programs(0) - 1)
    def _globals():
        mean = acc_ref[...] * (1.0 / N)
        gh = jnp.maximum(
            jnp.dot(u_ref[...], wgu_ref[...], preferred_element_type=jnp.float32)
            + jnp.dot(mean, wgm_ref[...], preferred_element_type=jnp.float32)
            + bg1_ref[...],
            0.0)
        go_ref[...] = jnp.dot(gh, wg2_ref[...], preferred_element_type=jnp.float32) + bg2_ref[...]


def _node_global(x, p, u, wnx, wna, wnu, bn1, wn2, bn2, wgu, wgm, bg1, wg2, bg2):
    bn = 2000
    return pl.pallas_call(
        _node_body,
        grid=(N // bn,),
        in_specs=[
            pl.BlockSpec((bn, DN), lambda i: (i, 0)),
            pl.BlockSpec((NC, bn, DE), lambda i: (0, i, 0)),
            pl.BlockSpec((1, DG), lambda i: (0, 0)),
            pl.BlockSpec((DN, L), lambda i: (0, 0)),
            pl.BlockSpec((DE, L), lambda i: (0, 0)),
            pl.BlockSpec((DG, L), lambda i: (0, 0)),
            pl.BlockSpec((1, L), lambda i: (0, 0)),
            pl.BlockSpec((L, DN), lambda i: (0, 0)),
            pl.BlockSpec((1, DN), lambda i: (0, 0)),
            pl.BlockSpec((DG, L), lambda i: (0, 0)),
            pl.BlockSpec((DN, L), lambda i: (0, 0)),
            pl.BlockSpec((1, L), lambda i: (0, 0)),
            pl.BlockSpec((L, DG), lambda i: (0, 0)),
            pl.BlockSpec((1, DG), lambda i: (0, 0)),
        ],
        out_specs=[
            pl.BlockSpec((bn, DN), lambda i: (i, 0)),
            pl.BlockSpec((1, DG), lambda i: (0, 0)),
        ],
        out_shape=[
            jax.ShapeDtypeStruct((N, DN), jnp.float32),
            jax.ShapeDtypeStruct((1, DG), jnp.float32),
        ],
        scratch_shapes=[pltpu.VMEM((1, DN), jnp.float32)],
    )(x, p, u, wnx, wna, wnu, bn1, wn2, bn2, wgu, wgm, bg1, wg2, bg2)


def kernel(x, edge_index, edge_attr, global_attr,
           We1, be1, We2, be2,
           Wn1, bn1, Wn2, bn2,
           Wg1, bg1, Wg2, bg2):
    row2d = edge_index[0].reshape(NCHUNK, CH)
    col2d = edge_index[1].reshape(NCHUNK, CH)

    xs, xd = _node_tables(
        x, We1[:DN], We1[DN:2 * DN], global_attr,
        We1[2 * DN + DE:], be1.reshape(1, L))

    gs, gd = _gather_kernel(xs, xd, row2d, col2d)

    eo_t = _edge_mlp(
        gs, gd, edge_attr.T, We1[2 * DN:2 * DN + DE], We2,
        be2.reshape(DE, 1))
    edge_out = eo_t.T

    zeros = jnp.zeros((ROWS_PER_TILE, DE), jnp.float32)
    partials = _scatter_kernel(eo_t, col2d, zeros)

    x_out, global_out = _node_global(
        x, partials, global_attr,
        Wn1[:DN], Wn1[DN:DN + DE], Wn1[DN + DE:], bn1.reshape(1, L),
        Wn2, bn2.reshape(1, DN),
        Wg1[:DG], Wg1[DG:], bg1.reshape(1, L),
        Wg2, bg2.reshape(1, DG))

    return (x_out, edge_out, global_out)
